# baseline (speedup 1.0000x reference)
import functools

import jax
import jax.numpy as jnp
from jax import lax
from jax.experimental import pallas as pl
from jax.experimental.pallas import tpu as pltpu
from jax.experimental.pallas import tpu_sc as plsc


N_LEVELS = 4
_BB = 256


def _level_body(r_ref, q_ref, w_ref, idx_ref, rout_ref, wsq_ref):
    D = r_ref.shape[1]
    r = r_ref[...] - q_ref[:, :D]
    _dist_argmin(r, w_ref, idx_ref, rout_ref, wsq_ref)


def _level0_body(r_ref, w_ref, idx_ref, rout_ref, wsq_ref):
    _dist_argmin(r_ref[...], w_ref, idx_ref, rout_ref, wsq_ref)


def _dist_argmin(r, w_ref, idx_ref, rout_ref, wsq_ref):
    bB, D = r.shape
    K = w_ref.shape[0]
    iota_k = jax.lax.broadcasted_iota(jnp.int32, (1, K), 1)

    @pl.when(pl.program_id(0) == 0)
    def _prep():
        W = w_ref[...]
        wsq_ref[...] = jnp.sum(W * W, axis=1)

    W = w_ref[...]
    w_sq = wsq_ref[...][None, :]
    r_sq = jnp.sum(r * r, axis=1, keepdims=True)
    s = jax.lax.dot_general(r, W, (((1,), (1,)), ((), ())),
                            preferred_element_type=jnp.float32)
    dist = r_sq - 2.0 * s + w_sq
    m = jnp.min(dist, axis=1, keepdims=True)
    idx = jnp.min(jnp.where(dist == m, iota_k, K), axis=1)
    idx_ref[...] = idx
    rout_ref[...] = r


def _tc_level(r, q, W, lvl):
    B, D = r.shape
    K = W.shape[0]
    bB = _BB
    grid = (B // bB,)
    out_shapes = (
        jax.ShapeDtypeStruct((B,), jnp.int32),
        jax.ShapeDtypeStruct((B, D), jnp.float32),
    )
    out_specs = (
        pl.BlockSpec((bB,), lambda i: (i,)),
        pl.BlockSpec((bB, D), lambda i: (i, 0)),
    )
    rq_spec = pl.BlockSpec((bB, D), lambda i: (i, 0))
    qpad_spec = pl.BlockSpec((bB, 2 * D), lambda i: (i, 0))
    cb_spec = pl.BlockSpec((K, D), lambda i: (0, 0))
    scratch = [pltpu.VMEM((K,), jnp.float32)]
    if lvl == 0:
        return pl.pallas_call(
            _level0_body, grid=grid,
            in_specs=[rq_spec, cb_spec],
            out_specs=out_specs, out_shape=out_shapes,
            scratch_shapes=scratch,
        )(r, W)
    return pl.pallas_call(
        _level_body, grid=grid,
        in_specs=[rq_spec, qpad_spec, cb_spec],
        out_specs=out_specs, out_shape=out_shapes,
        scratch_shapes=scratch,
    )(r, q, W)


def _make_sc_gather(K, D, B):
    info = plsc.get_sparse_core_info()
    NW = info.num_cores * info.num_subcores
    b_per_w = B // NW
    n_idx = 128
    chunks = b_per_w // n_idx
    n_outer = 2
    rows_per_outer = b_per_w // n_outer
    g_per_outer = rows_per_outer // n_idx
    mesh = plsc.VectorSubcoreMesh(core_axis_name="c", subcore_axis_name="s")

    @functools.partial(
        pl.kernel, mesh=mesh,
        out_type=jax.ShapeDtypeStruct((B, 2 * D), jnp.float32),
        scratch_types=[
            pltpu.VMEM((chunks, n_idx), jnp.int32),
            pltpu.VMEM((rows_per_outer, 2 * D), jnp.float32),
            pltpu.SemaphoreType.DMA,
        ],
    )
    def _gather(table_hbm, idx_hbm, out_hbm, idx_v, rows_v, sem):
        wid = lax.axis_index("s") * info.num_cores + lax.axis_index("c")
        base = wid * b_per_w
        pltpu.sync_copy(idx_hbm.at[pl.ds(wid * chunks, chunks)], idx_v)
        for o in range(n_outer):
            copies = [
                pltpu.async_copy(
                    table_hbm.at[idx_v.at[o * g_per_outer + j]],
                    rows_v.at[pl.ds(j * n_idx, n_idx)], sem)
                for j in range(g_per_outer)]
            for c in copies:
                c.wait()
            pltpu.sync_copy(
                rows_v,
                out_hbm.at[pl.ds(base + o * rows_per_outer, rows_per_outer)])

    return _gather


def _zq_body(q0_ref, q1_ref, q2_ref, q3_ref, zq_ref):
    D = zq_ref.shape[1]
    zq_ref[...] = ((q0_ref[:, :D] + q1_ref[:, :D]) + q2_ref[:, :D]) + q3_ref[:, :D]


def _zq(q0, q1, q2, q3):
    B, D2 = q0.shape
    D = D2 // 2
    bB = 1024
    qspec = pl.BlockSpec((bB, D2), lambda i: (i, 0))
    return pl.pallas_call(
        _zq_body, grid=(B // bB,),
        in_specs=[qspec] * 4,
        out_specs=pl.BlockSpec((bB, D), lambda i: (i, 0)),
        out_shape=jax.ShapeDtypeStruct((B, D), jnp.float32),
    )(q0, q1, q2, q3)


def kernel(h, cb0, cb1, cb2, cb3):
    B, D = h.shape
    K = cb0.shape[0]
    gather = _make_sc_gather(K, D, B)
    cbs = (cb0, cb1, cb2, cb3)
    rs, qs, idxs = [], [], []
    r, q = h, None
    for lvl in range(N_LEVELS):
        idx, r = _tc_level(r, q, cbs[lvl], lvl)
        wpad = jnp.pad(cbs[lvl], ((0, 0), (0, D)))
        q = gather(wpad, idx.reshape(B // 128, 128))
        rs.append(r)
        qs.append(q)
        idxs.append(idx)
    z_q = _zq(*qs)
    sids = jnp.stack(idxs, axis=1)
    residuals = jnp.stack(rs, axis=1)
    quantized = jnp.stack([q[:, :D] for q in qs], axis=1)
    active_mask = jnp.ones((N_LEVELS,), dtype=h.dtype)
    return (z_q, sids, residuals, quantized, active_mask)


# speedup vs baseline: 1.2520x; 1.2520x over previous
import jax
import jax.numpy as jnp
from jax.experimental import pallas as pl
from jax.experimental.pallas import tpu as pltpu


N_LEVELS = 4


def _rvq_body(h_ref, w0_ref, w1_ref, w2_ref, w3_ref,
              zq_ref, sids_ref, res_ref, quant_ref, wsq_ref, wcat_ref):
    r = h_ref[...]
    bB, D = r.shape
    K = w0_ref.shape[0]
    iota_k = jax.lax.broadcasted_iota(jnp.int32, (1, K), 1)

    @pl.when(pl.program_id(0) == 0)
    def _prep():
        for lvl, w_ref in enumerate((w0_ref, w1_ref, w2_ref, w3_ref)):
            W = w_ref[...]
            wsq_ref[lvl, :] = jnp.sum(W * W, axis=1)
            w_hi = W.astype(jnp.bfloat16).astype(jnp.float32)
            wcat_ref[lvl, :, :D] = w_hi
            wcat_ref[lvl, :, D:] = W - w_hi

    zq = jnp.zeros_like(r)
    idx_cols = []
    for lvl, w_ref in enumerate((w0_ref, w1_ref, w2_ref, w3_ref)):
        W = w_ref[...]
        w_sq = wsq_ref[lvl, :][None, :]
        r_sq = jnp.sum(r * r, axis=1, keepdims=True)
        s = jax.lax.dot_general(r, W, (((1,), (1,)), ((), ())),
                                preferred_element_type=jnp.float32)
        dist = r_sq - 2.0 * s + w_sq
        m = jnp.min(dist, axis=1, keepdims=True)
        idx = jnp.min(jnp.where(dist == m, iota_k, K), axis=1)

        onehot = (iota_k == idx[:, None]).astype(jnp.float32)
        q_cat = jax.lax.dot_general(onehot, wcat_ref[lvl],
                                    (((1,), (0,)), ((), ())),
                                    preferred_element_type=jnp.float32)
        q = q_cat[:, :D] + q_cat[:, D:]

        res_ref[:, lvl, :] = r
        quant_ref[:, lvl, :] = q
        idx_cols.append(idx)
        zq = zq + q
        r = r - q

    sids_ref[...] = jnp.stack(idx_cols, axis=1)
    zq_ref[...] = zq


@jax.jit
def _rvq(h, cb0, cb1, cb2, cb3):
    B, D = h.shape
    K = cb0.shape[0]
    bB = 512
    grid = (B // bB,)
    cb_spec = pl.BlockSpec((K, D), lambda i: (0, 0))
    out_shapes = (
        jax.ShapeDtypeStruct((B, D), jnp.float32),
        jax.ShapeDtypeStruct((B, N_LEVELS), jnp.int32),
        jax.ShapeDtypeStruct((B, N_LEVELS, D), jnp.float32),
        jax.ShapeDtypeStruct((B, N_LEVELS, D), jnp.float32),
    )
    out_specs = (
        pl.BlockSpec((bB, D), lambda i: (i, 0)),
        pl.BlockSpec((bB, N_LEVELS), lambda i: (i, 0)),
        pl.BlockSpec((bB, N_LEVELS, D), lambda i: (i, 0, 0)),
        pl.BlockSpec((bB, N_LEVELS, D), lambda i: (i, 0, 0)),
    )
    in_specs = [
        pl.BlockSpec((bB, D), lambda i: (i, 0)),
        cb_spec, cb_spec, cb_spec, cb_spec,
    ]
    return pl.pallas_call(
        _rvq_body,
        grid=grid,
        in_specs=in_specs,
        out_specs=out_specs,
        out_shape=out_shapes,
        scratch_shapes=[
            pltpu.VMEM((N_LEVELS, K), jnp.float32),
            pltpu.VMEM((N_LEVELS, K, 2 * D), jnp.float32),
        ],
    )(h, cb0, cb1, cb2, cb3)


def kernel(h, cb0, cb1, cb2, cb3):
    z_q, sids, residuals, quantized = _rvq(h, cb0, cb1, cb2, cb3)
    active_mask = jnp.ones((N_LEVELS,), dtype=h.dtype)
    return (z_q, sids, residuals, quantized, active_mask)


# speedup vs baseline: 1.2910x; 1.0311x over previous
import jax
import jax.numpy as jnp
from jax.experimental import pallas as pl
from jax.experimental.pallas import tpu as pltpu


N_LEVELS = 4
_KC = 256


def _rvq_body(h_ref, w0_ref, w1_ref, w2_ref, w3_ref,
              zq_ref, sids_ref, res_ref, quant_ref,
              wsq_ref, wcat_ref, dist_ref):
    r = h_ref[...]
    bB, D = r.shape
    K = w0_ref.shape[0]
    NC = K // _KC
    iota_k = jax.lax.broadcasted_iota(jnp.int32, (1, K), 1)

    @pl.when(pl.program_id(0) == 0)
    def _prep():
        for lvl, w_ref in enumerate((w0_ref, w1_ref, w2_ref, w3_ref)):
            W = w_ref[...]
            wsq_ref[lvl, :] = jnp.sum(W * W, axis=1)
            w_hi = W.astype(jnp.bfloat16).astype(jnp.float32)
            wcat_ref[lvl, :, :D] = w_hi
            wcat_ref[lvl, :, D:] = W - w_hi

    zq = jnp.zeros_like(r)
    idx_cols = []
    for lvl, w_ref in enumerate((w0_ref, w1_ref, w2_ref, w3_ref)):
        r_sq = jnp.sum(r * r, axis=1, keepdims=True)
        m = None
        for c in range(NC):
            Wc = w_ref[pl.ds(c * _KC, _KC), :]
            w_sq_c = wsq_ref[lvl, pl.ds(c * _KC, _KC)][None, :]
            s_c = jax.lax.dot_general(r, Wc, (((1,), (1,)), ((), ())),
                                      preferred_element_type=jnp.float32)
            dist_c = r_sq - 2.0 * s_c + w_sq_c
            dist_ref[:, pl.ds(c * _KC, _KC)] = dist_c
            mc = jnp.min(dist_c, axis=1, keepdims=True)
            m = mc if m is None else jnp.minimum(m, mc)
        dist = dist_ref[...]
        idx = jnp.min(jnp.where(dist == m, iota_k, K), axis=1)

        onehot = (iota_k == idx[:, None]).astype(jnp.float32)
        q_cat = jax.lax.dot_general(onehot, wcat_ref[lvl],
                                    (((1,), (0,)), ((), ())),
                                    preferred_element_type=jnp.float32)
        q = q_cat[:, :D] + q_cat[:, D:]

        res_ref[:, lvl, :] = r
        quant_ref[:, lvl, :] = q
        idx_cols.append(idx)
        zq = zq + q
        r = r - q

    sids_ref[...] = jnp.stack(idx_cols, axis=1)
    zq_ref[...] = zq


@jax.jit
def _rvq(h, cb0, cb1, cb2, cb3):
    B, D = h.shape
    K = cb0.shape[0]
    bB = 512
    grid = (B // bB,)
    cb_spec = pl.BlockSpec((K, D), lambda i: (0, 0))
    out_shapes = (
        jax.ShapeDtypeStruct((B, D), jnp.float32),
        jax.ShapeDtypeStruct((B, N_LEVELS), jnp.int32),
        jax.ShapeDtypeStruct((B, N_LEVELS, D), jnp.float32),
        jax.ShapeDtypeStruct((B, N_LEVELS, D), jnp.float32),
    )
    out_specs = (
        pl.BlockSpec((bB, D), lambda i: (i, 0)),
        pl.BlockSpec((bB, N_LEVELS), lambda i: (i, 0)),
        pl.BlockSpec((bB, N_LEVELS, D), lambda i: (i, 0, 0)),
        pl.BlockSpec((bB, N_LEVELS, D), lambda i: (i, 0, 0)),
    )
    in_specs = [
        pl.BlockSpec((bB, D), lambda i: (i, 0)),
        cb_spec, cb_spec, cb_spec, cb_spec,
    ]
    return pl.pallas_call(
        _rvq_body,
        grid=grid,
        in_specs=in_specs,
        out_specs=out_specs,
        out_shape=out_shapes,
        scratch_shapes=[
            pltpu.VMEM((N_LEVELS, K), jnp.float32),
            pltpu.VMEM((N_LEVELS, K, 2 * D), jnp.float32),
            pltpu.VMEM((512, K), jnp.float32),
        ],
    )(h, cb0, cb1, cb2, cb3)


def kernel(h, cb0, cb1, cb2, cb3):
    z_q, sids, residuals, quantized = _rvq(h, cb0, cb1, cb2, cb3)
    active_mask = jnp.ones((N_LEVELS,), dtype=h.dtype)
    return (z_q, sids, residuals, quantized, active_mask)


# speedup vs baseline: 1.3419x; 1.0394x over previous
import jax
import jax.numpy as jnp
from jax.experimental import pallas as pl
from jax.experimental.pallas import tpu as pltpu


N_LEVELS = 4
_KC = 512


def _rvq_body(h_ref, w0_ref, w1_ref, w2_ref, w3_ref,
              zq_ref, sids_ref, res_ref, quant_ref,
              wsq_ref, wcat_ref):
    r = h_ref[...]
    bB, D = r.shape
    K = w0_ref.shape[0]
    NC = K // _KC
    iota_k = jax.lax.broadcasted_iota(jnp.int32, (1, K), 1)

    @pl.when(pl.program_id(0) == 0)
    def _prep():
        for lvl, w_ref in enumerate((w0_ref, w1_ref, w2_ref, w3_ref)):
            W = w_ref[...]
            wsq_ref[lvl, :] = jnp.sum(W * W, axis=1)
            w_hi = W.astype(jnp.bfloat16).astype(jnp.float32)
            wcat_ref[lvl, :, :D] = w_hi
            wcat_ref[lvl, :, D:] = W - w_hi

    zq = jnp.zeros_like(r)
    idx_cols = []
    for lvl, w_ref in enumerate((w0_ref, w1_ref, w2_ref, w3_ref)):
        r_sq = jnp.sum(r * r, axis=1, keepdims=True)
        iota_c = jax.lax.broadcasted_iota(jnp.int32, (1, _KC), 1)
        m = None
        idx = None
        for c in range(NC):
            Wc = w_ref[pl.ds(c * _KC, _KC), :]
            w_sq_c = wsq_ref[lvl, pl.ds(c * _KC, _KC)][None, :]
            s_c = jax.lax.dot_general(r, Wc, (((1,), (1,)), ((), ())),
                                      preferred_element_type=jnp.float32)
            dist_c = r_sq - 2.0 * s_c + w_sq_c
            mc = jnp.min(dist_c, axis=1, keepdims=True)
            ic = jnp.min(jnp.where(dist_c == mc, iota_c + c * _KC, K),
                         axis=1)
            if m is None:
                m, idx = mc, ic
            else:
                idx = jnp.where(mc[:, 0] < m[:, 0], ic, idx)
                m = jnp.minimum(m, mc)

        onehot = (iota_k == idx[:, None]).astype(jnp.float32)
        q_cat = jax.lax.dot_general(onehot, wcat_ref[lvl],
                                    (((1,), (0,)), ((), ())),
                                    preferred_element_type=jnp.float32)
        q = q_cat[:, :D] + q_cat[:, D:]

        res_ref[:, lvl, :] = r
        quant_ref[:, lvl, :] = q
        idx_cols.append(idx)
        zq = zq + q
        r = r - q

    sids_ref[...] = jnp.stack(idx_cols, axis=1)
    zq_ref[...] = zq


@jax.jit
def _rvq(h, cb0, cb1, cb2, cb3):
    B, D = h.shape
    K = cb0.shape[0]
    bB = 512
    grid = (B // bB,)
    cb_spec = pl.BlockSpec((K, D), lambda i: (0, 0))
    out_shapes = (
        jax.ShapeDtypeStruct((B, D), jnp.float32),
        jax.ShapeDtypeStruct((B, N_LEVELS), jnp.int32),
        jax.ShapeDtypeStruct((B, N_LEVELS, D), jnp.float32),
        jax.ShapeDtypeStruct((B, N_LEVELS, D), jnp.float32),
    )
    out_specs = (
        pl.BlockSpec((bB, D), lambda i: (i, 0)),
        pl.BlockSpec((bB, N_LEVELS), lambda i: (i, 0)),
        pl.BlockSpec((bB, N_LEVELS, D), lambda i: (i, 0, 0)),
        pl.BlockSpec((bB, N_LEVELS, D), lambda i: (i, 0, 0)),
    )
    in_specs = [
        pl.BlockSpec((bB, D), lambda i: (i, 0)),
        cb_spec, cb_spec, cb_spec, cb_spec,
    ]
    return pl.pallas_call(
        _rvq_body,
        grid=grid,
        in_specs=in_specs,
        out_specs=out_specs,
        out_shape=out_shapes,
        scratch_shapes=[
            pltpu.VMEM((N_LEVELS, K), jnp.float32),
            pltpu.VMEM((N_LEVELS, K, 2 * D), jnp.float32),
        ],
    )(h, cb0, cb1, cb2, cb3)


def kernel(h, cb0, cb1, cb2, cb3):
    z_q, sids, residuals, quantized = _rvq(h, cb0, cb1, cb2, cb3)
    active_mask = jnp.ones((N_LEVELS,), dtype=h.dtype)
    return (z_q, sids, residuals, quantized, active_mask)


# speedup vs baseline: 1.3958x; 1.0401x over previous
import jax
import jax.numpy as jnp
from jax.experimental import pallas as pl
from jax.experimental.pallas import tpu as pltpu


N_LEVELS = 4
_KC = 512


def _rvq_body(h_ref, w0_ref, w1_ref, w2_ref, w3_ref,
              zq_ref, sids_ref, res_ref, quant_ref,
              wsq_ref, wcat_ref):
    r = h_ref[...]
    bB, D = r.shape
    K = w0_ref.shape[0]
    NC = K // _KC
    iota_k = jax.lax.broadcasted_iota(jnp.int32, (1, K), 1)

    @pl.when(pl.program_id(0) == 0)
    def _prep():
        for lvl, w_ref in enumerate((w0_ref, w1_ref, w2_ref, w3_ref)):
            W = w_ref[...]
            wsq_ref[lvl, :] = jnp.sum(W * W, axis=1)
            w_hi = W.astype(jnp.bfloat16).astype(jnp.float32)
            wcat_ref[lvl, :, :D] = w_hi
            wcat_ref[lvl, :, D:] = W - w_hi

    zq = jnp.zeros_like(r)
    idx_cols = []
    for lvl, w_ref in enumerate((w0_ref, w1_ref, w2_ref, w3_ref)):
        r_sq = jnp.sum(r * r, axis=1, keepdims=True)
        iota_c = jax.lax.broadcasted_iota(jnp.int32, (1, _KC), 1)
        m = None
        idx = None
        for c in range(NC):
            Wc = w_ref[pl.ds(c * _KC, _KC), :]
            w_sq_c = wsq_ref[lvl, pl.ds(c * _KC, _KC)][None, :]
            s_c = jax.lax.dot_general(r, Wc, (((1,), (1,)), ((), ())),
                                      preferred_element_type=jnp.float32)
            dist_c = r_sq - 2.0 * s_c + w_sq_c
            mc = jnp.min(dist_c, axis=1, keepdims=True)
            ic = jnp.min(jnp.where(dist_c == mc, iota_c + c * _KC, K),
                         axis=1)
            if m is None:
                m, idx = mc, ic
            else:
                idx = jnp.where(mc[:, 0] < m[:, 0], ic, idx)
                m = jnp.minimum(m, mc)

        onehot = (iota_k == idx[:, None]).astype(jnp.float32)
        q_cat = jax.lax.dot_general(onehot, wcat_ref[lvl],
                                    (((1,), (0,)), ((), ())),
                                    preferred_element_type=jnp.float32)
        q = q_cat[:, :D] + q_cat[:, D:]

        res_ref[:, lvl, :] = r
        quant_ref[:, lvl, :] = q
        idx_cols.append(idx)
        zq = zq + q
        r = r - q

    sids_ref[...] = jnp.stack(idx_cols, axis=1)
    zq_ref[...] = zq


@jax.jit
def _rvq(h, cb0, cb1, cb2, cb3):
    B, D = h.shape
    K = cb0.shape[0]
    bB = 1024
    grid = (B // bB,)
    cb_spec = pl.BlockSpec((K, D), lambda i: (0, 0))
    out_shapes = (
        jax.ShapeDtypeStruct((B, D), jnp.float32),
        jax.ShapeDtypeStruct((B, N_LEVELS), jnp.int32),
        jax.ShapeDtypeStruct((B, N_LEVELS, D), jnp.float32),
        jax.ShapeDtypeStruct((B, N_LEVELS, D), jnp.float32),
    )
    out_specs = (
        pl.BlockSpec((bB, D), lambda i: (i, 0)),
        pl.BlockSpec((bB, N_LEVELS), lambda i: (i, 0)),
        pl.BlockSpec((bB, N_LEVELS, D), lambda i: (i, 0, 0)),
        pl.BlockSpec((bB, N_LEVELS, D), lambda i: (i, 0, 0)),
    )
    in_specs = [
        pl.BlockSpec((bB, D), lambda i: (i, 0)),
        cb_spec, cb_spec, cb_spec, cb_spec,
    ]
    return pl.pallas_call(
        _rvq_body,
        grid=grid,
        in_specs=in_specs,
        out_specs=out_specs,
        out_shape=out_shapes,
        scratch_shapes=[
            pltpu.VMEM((N_LEVELS, K), jnp.float32),
            pltpu.VMEM((N_LEVELS, K, 2 * D), jnp.float32),
        ],
    )(h, cb0, cb1, cb2, cb3)


def kernel(h, cb0, cb1, cb2, cb3):
    z_q, sids, residuals, quantized = _rvq(h, cb0, cb1, cb2, cb3)
    active_mask = jnp.ones((N_LEVELS,), dtype=h.dtype)
    return (z_q, sids, residuals, quantized, active_mask)


# speedup vs baseline: 1.4351x; 1.0281x over previous
import jax
import jax.numpy as jnp
from jax.experimental import pallas as pl
from jax.experimental.pallas import tpu as pltpu


N_LEVELS = 4
_KC = 512


def _rvq_body(h_ref, w0_ref, w1_ref, w2_ref, w3_ref,
              zq_ref, sids_ref, res_ref, quant_ref,
              wsq_ref, wcat_ref):
    r = h_ref[...]
    bB, D = r.shape
    K = w0_ref.shape[0]
    NC = K // _KC
    iota_k = jax.lax.broadcasted_iota(jnp.int32, (1, K), 1)

    @pl.when(pl.program_id(0) == 0)
    def _prep():
        for lvl, w_ref in enumerate((w0_ref, w1_ref, w2_ref, w3_ref)):
            W = w_ref[...]
            wsq_ref[lvl, :] = jnp.sum(W * W, axis=1)
            w_hi = W.astype(jnp.bfloat16).astype(jnp.float32)
            wcat_ref[lvl, :, :D] = w_hi
            wcat_ref[lvl, :, D:] = W - w_hi

    zq = jnp.zeros_like(r)
    idx_cols = []
    for lvl, w_ref in enumerate((w0_ref, w1_ref, w2_ref, w3_ref)):
        r_sq = jnp.sum(r * r, axis=1, keepdims=True)
        iota_c = jax.lax.broadcasted_iota(jnp.int32, (1, _KC), 1)
        m = None
        idx = None
        for c in range(NC):
            Wc = w_ref[pl.ds(c * _KC, _KC), :]
            w_sq_c = wsq_ref[lvl, pl.ds(c * _KC, _KC)][None, :]
            s_c = jax.lax.dot_general(r, Wc, (((1,), (1,)), ((), ())),
                                      preferred_element_type=jnp.float32)
            dist_c = r_sq - 2.0 * s_c + w_sq_c
            mc = jnp.min(dist_c, axis=1, keepdims=True)
            ic = jnp.min(jnp.where(dist_c == mc, iota_c + c * _KC, K),
                         axis=1)
            if m is None:
                m, idx = mc, ic
            else:
                idx = jnp.where(mc[:, 0] < m[:, 0], ic, idx)
                m = jnp.minimum(m, mc)

        onehot = (iota_k == idx[:, None]).astype(jnp.float32)
        q_cat = jax.lax.dot_general(onehot, wcat_ref[lvl],
                                    (((1,), (0,)), ((), ())),
                                    preferred_element_type=jnp.float32)
        q = q_cat[:, :D] + q_cat[:, D:]

        res_ref[:, lvl, :] = r
        quant_ref[:, lvl, :] = q
        idx_cols.append(idx)
        zq = zq + q
        r = r - q

    sids_ref[...] = jnp.stack(idx_cols, axis=1)
    zq_ref[...] = zq


@jax.jit
def _rvq(h, cb0, cb1, cb2, cb3):
    B, D = h.shape
    K = cb0.shape[0]
    bB = 2048
    grid = (B // bB,)
    cb_spec = pl.BlockSpec((K, D), lambda i: (0, 0))
    out_shapes = (
        jax.ShapeDtypeStruct((B, D), jnp.float32),
        jax.ShapeDtypeStruct((B, N_LEVELS), jnp.int32),
        jax.ShapeDtypeStruct((B, N_LEVELS, D), jnp.float32),
        jax.ShapeDtypeStruct((B, N_LEVELS, D), jnp.float32),
    )
    out_specs = (
        pl.BlockSpec((bB, D), lambda i: (i, 0)),
        pl.BlockSpec((bB, N_LEVELS), lambda i: (i, 0)),
        pl.BlockSpec((bB, N_LEVELS, D), lambda i: (i, 0, 0)),
        pl.BlockSpec((bB, N_LEVELS, D), lambda i: (i, 0, 0)),
    )
    in_specs = [
        pl.BlockSpec((bB, D), lambda i: (i, 0)),
        cb_spec, cb_spec, cb_spec, cb_spec,
    ]
    return pl.pallas_call(
        _rvq_body,
        grid=grid,
        in_specs=in_specs,
        out_specs=out_specs,
        out_shape=out_shapes,
        scratch_shapes=[
            pltpu.VMEM((N_LEVELS, K), jnp.float32),
            pltpu.VMEM((N_LEVELS, K, 2 * D), jnp.float32),
        ],
    )(h, cb0, cb1, cb2, cb3)


def kernel(h, cb0, cb1, cb2, cb3):
    z_q, sids, residuals, quantized = _rvq(h, cb0, cb1, cb2, cb3)
    active_mask = jnp.ones((N_LEVELS,), dtype=h.dtype)
    return (z_q, sids, residuals, quantized, active_mask)
